# baseline (speedup 1.0000x reference)
import functools

import jax
import jax.numpy as jnp
from jax import lax
from jax.experimental import pallas as pl
from jax.experimental.pallas import tpu as pltpu
from jax.experimental.pallas import tpu_sc as plsc

N = 10000
E = 320000
NPAD = 10240
ROWS = NPAD // 128
NS = 16
EC = E // NS
RPS = ROWS // NS
INNER = RPS * 128
B = 1024
ACT = 48


def _newton_rsqrt(d):
    i = plsc.bitcast(d, jnp.int32)
    i = jnp.full((16,), 0x5F3759DF, jnp.int32) - lax.shift_right_logical(i, 1)
    y = plsc.bitcast(i, jnp.float32)
    for _ in range(3):
        y = y * (1.5 - 0.5 * d * y * y)
    return y


def _nidx(n):
    t = lax.bitwise_and(lax.shift_right_logical(n, 7), 15)
    inner = lax.bitwise_or(
        lax.bitwise_and(lax.shift_right_logical(n, 4),
                        jnp.full((16,), ~127, jnp.int32)),
        lax.bitwise_and(n, 127))
    return [t, inner]


def _make_sc_fused():
    mesh = plsc.VectorSubcoreMesh(core_axis_name="c", subcore_axis_name="s")

    @functools.partial(
        pl.kernel,
        out_type=(
            jax.ShapeDtypeStruct((NS, INNER), jnp.float32),
            jax.ShapeDtypeStruct((NS, INNER), jnp.float32),
            jax.ShapeDtypeStruct((NS, INNER), jnp.float32),
            jax.ShapeDtypeStruct((NS, INNER), jnp.float32),
            jax.ShapeDtypeStruct((NS, NS, INNER), jnp.float32),
        ),
        mesh=mesh,
        compiler_params=pltpu.CompilerParams(needs_layout_passes=False),
        scratch_types=[
            pltpu.VMEM((EC,), jnp.int32),
            pltpu.VMEM((EC,), jnp.int32),
            pltpu.VMEM((NS, INNER), jnp.float32),
            pltpu.VMEM((NS, INNER), jnp.float32),
            pltpu.VMEM((NS, INNER), jnp.float32),
            pltpu.VMEM((NS, INNER), jnp.float32),
            pltpu.VMEM((NS, INNER), jnp.float32),
            pltpu.VMEM((INNER,), jnp.float32),
            pltpu.VMEM((INNER,), jnp.float32),
            pltpu.VMEM((INNER,), jnp.float32),
            pltpu.SemaphoreType.DMA,
            pltpu.SemaphoreType.DMA,
            pltpu.SemaphoreType.DMA,
            pltpu.SemaphoreType.DMA,
        ],
    )
    def sc_fused(dst_hbm, src_hbm, zeros_hbm, u_hbm, v_hbm, dstage_hbm,
                 wstage_hbm, parts_hbm,
                 dst_v, src_v, table_v, accA, accB, accC, mbuf_v, dslice_v,
                 ova_v, ovb_v, sem_d, sem_s, sem_z, sem_m):
        c = lax.axis_index("c")
        s = lax.axis_index("s")

        @pl.when(c == 0)
        def _():
            base = s * EC
            cp_d = pltpu.async_copy(dst_hbm.at[pl.ds(base, EC)], dst_v, sem_d)
            cp_s = pltpu.async_copy(src_hbm.at[pl.ds(base, EC)], src_v, sem_s)
            cp_a = pltpu.async_copy(zeros_hbm, accA, sem_z)
            cp_b = pltpu.async_copy(zeros_hbm, accB, sem_z)
            cp_c = pltpu.async_copy(zeros_hbm, accC, sem_z)
            cp_a.wait()
            cp_b.wait()
            cp_c.wait()

            def merge_read():
                cps = [pltpu.async_copy(parts_hbm.at[k, s], mbuf_v.at[k],
                                        sem_m)
                       for k in range(NS)]
                for cp in cps:
                    cp.wait()

            def merged_chunk(q):
                m = mbuf_v[0, pl.ds(q * 16, 16)]
                for k in range(1, NS):
                    m = m + mbuf_v[k, pl.ds(q * 16, 16)]
                return m

            cp_d.wait()
            ones16 = jnp.ones((16,), jnp.float32)

            def body1(i, carry):
                si = dst_v[pl.ds(i * 16, 16)]
                plsc.addupdate_scatter(accA, _nidx(si), ones16)
                return carry

            lax.fori_loop(0, EC // 16, body1, 0, unroll=8)
            pltpu.sync_copy(accA, parts_hbm.at[s])
            plsc.subcore_barrier()

            merge_read()
            for q in range(INNER // 16):
                d = merged_chunk(q) + 1.0
                dslice_v[pl.ds(q * 16, 16)] = _newton_rsqrt(d)
            pltpu.sync_copy(dslice_v, dstage_hbm.at[s])
            plsc.subcore_barrier()
            pltpu.sync_copy(dstage_hbm, table_v)

            cp_s.wait()

            def body2(acc):
                def _body(i, carry):
                    gi = dst_v[pl.ds(i * 16, 16)]
                    val = plsc.load_gather(table_v, _nidx(gi))
                    si = src_v[pl.ds(i * 16, 16)]
                    plsc.addupdate_scatter(acc, _nidx(si), val)
                    return carry
                return _body

            lax.fori_loop(0, EC // 16, body2(accB), 0, unroll=8)
            pltpu.sync_copy(accB, parts_hbm.at[s])
            plsc.subcore_barrier()

            merge_read()
            for q in range(INNER // 16):
                dv = dslice_v[pl.ds(q * 16, 16)]
                vv = dv * (merged_chunk(q) + dv)
                ova_v[pl.ds(q * 16, 16)] = vv
                ovb_v[pl.ds(q * 16, 16)] = dv * vv
            pltpu.sync_copy(ova_v, v_hbm.at[s])
            pltpu.sync_copy(ovb_v, wstage_hbm.at[s])
            plsc.subcore_barrier()
            pltpu.sync_copy(wstage_hbm, table_v)

            lax.fori_loop(0, EC // 16, body2(accC), 0, unroll=8)
            pltpu.sync_copy(accC, parts_hbm.at[s])
            plsc.subcore_barrier()

            merge_read()
            for q in range(INNER // 16):
                dv = dslice_v[pl.ds(q * 16, 16)]
                wv = ovb_v[pl.ds(q * 16, 16)]
                ova_v[pl.ds(q * 16, 16)] = dv * (merged_chunk(q) + wv)
            pltpu.sync_copy(ova_v, u_hbm.at[s])

    return sc_fused


_sc_fused = _make_sc_fused()


def _head_body(u_ref, v_ref, x3_ref,
               state_ref, action_ref,
               Wg1_ref, bg1_ref, Wg2_ref, bg2_ref,
               Wa0_ref, ba0_ref, Wa1_ref, ba1_ref, Wa2_ref, ba2_ref,
               Wc0_ref, bc0_ref, Wc1_ref, bc1_ref, Wc2_ref, bc2_ref,
               alp_ref, sval_ref, ent_ref):
    row = lax.broadcasted_iota(jnp.int32, (ROWS, 128), 0)
    col = lax.broadcasted_iota(jnp.int32, (ROWS, 128), 1)
    mask = (row * 128 + col) < N
    u = jnp.where(mask, u_ref[...], 0.0)
    sv = jnp.sum(jnp.where(mask, v_ref[...], 0.0))
    t = jnp.sum(x3_ref[...] * u[:, :, None], axis=(0, 1)).reshape(1, 128)
    g1 = jnp.dot(t, Wg1_ref[...], preferred_element_type=jnp.float32) \
        + sv * bg1_ref[...]
    g = jnp.dot(g1, Wg2_ref[...], preferred_element_type=jnp.float32) / N \
        + bg2_ref[...]

    st = state_ref[...]

    def mlp(W0_ref, b0_ref, W1_ref, b1_ref):
        h = jnp.tanh(
            jnp.dot(st, W0_ref[0:128, :], preferred_element_type=jnp.float32)
            + jnp.dot(g, W0_ref[128:256, :], preferred_element_type=jnp.float32)
            + b0_ref[...])
        return jnp.tanh(
            jnp.dot(h, W1_ref[...], preferred_element_type=jnp.float32)
            + b1_ref[...])

    ya = mlp(Wa0_ref, ba0_ref, Wa1_ref, ba1_ref)
    logits = jnp.dot(ya, Wa2_ref[...], preferred_element_type=jnp.float32) \
        + ba2_ref[...]
    m = jnp.max(logits, axis=1, keepdims=True)
    ex = jnp.exp(logits - m)
    ssum = jnp.sum(ex, axis=1, keepdims=True)
    logp = logits - m - jnp.log(ssum)
    probs = ex / ssum
    onehot = lax.broadcasted_iota(jnp.int32, (B, ACT), 1) == action_ref[...]
    alp_ref[...] = jnp.sum(jnp.where(onehot, logp, 0.0), axis=1, keepdims=True)
    ent_ref[...] = -jnp.sum(probs * logp, axis=1, keepdims=True)

    yc = mlp(Wc0_ref, bc0_ref, Wc1_ref, bc1_ref)
    sval_ref[...] = jnp.dot(yc, Wc2_ref[...], preferred_element_type=jnp.float32) \
        + bc2_ref[...]


def kernel(state, action, x, edge_index, W_g1, b_g1, W_g2, b_g2,
           Wa0, ba0, Wa1, ba1, Wa2, ba2, Wc0, bc0, Wc1, bc1, Wc2, bc2):
    src = edge_index[0]
    dst = edge_index[1]
    zeros_t = jnp.zeros((NS, INNER), jnp.float32)

    u3, v3, _, _, _ = _sc_fused(dst, src, zeros_t)
    u2d = jnp.transpose(u3.reshape(NS, RPS, 128), (1, 0, 2)).reshape(ROWS, 128)
    v2d = jnp.transpose(v3.reshape(NS, RPS, 128), (1, 0, 2)).reshape(ROWS, 128)

    x3 = jnp.pad(x, ((0, NPAD - N), (0, 0))).reshape(ROWS, 128, 128)
    action2 = action.astype(jnp.int32).reshape(B, 1)

    alp, sval, ent = pl.pallas_call(
        _head_body,
        out_shape=(jax.ShapeDtypeStruct((B, 1), jnp.float32),
                   jax.ShapeDtypeStruct((B, 1), jnp.float32),
                   jax.ShapeDtypeStruct((B, 1), jnp.float32)),
    )(u2d, v2d, x3, state, action2,
      W_g1, b_g1, W_g2, b_g2,
      Wa0, ba0, Wa1, ba1, Wa2, ba2,
      Wc0, bc0, Wc1, bc1, Wc2, bc2)

    return (alp[:, 0], sval, ent[:, 0])


# speedup vs baseline: 1.3039x; 1.3039x over previous
import functools

import jax
import jax.numpy as jnp
from jax import lax
from jax.experimental import pallas as pl
from jax.experimental.pallas import tpu as pltpu
from jax.experimental.pallas import tpu_sc as plsc

N = 10000
E = 320000
NPAD = 10240
ROWS = NPAD // 128
NC = 2
NS = 16
NW = NC * NS
CH = E // NW
B = 1024
ACT = 48


def _zero_acc(acc_v):
    zeros16 = jnp.zeros((16,), jnp.float32)

    def zbody(i, carry):
        acc_v[pl.ds(i * 16, 16)] = zeros16
        return carry

    lax.fori_loop(0, NPAD // 16, zbody, 0, unroll=8)


def _make_sc_deg():
    mesh = plsc.VectorSubcoreMesh(core_axis_name="c", subcore_axis_name="s")

    @functools.partial(
        pl.kernel,
        out_type=jax.ShapeDtypeStruct((NW, NPAD), jnp.float32),
        mesh=mesh,
        compiler_params=pltpu.CompilerParams(needs_layout_passes=False),
        scratch_types=[
            pltpu.VMEM((CH,), jnp.int32),
            pltpu.VMEM((NPAD,), jnp.float32),
            pltpu.SemaphoreType.DMA,
        ],
    )
    def sc_deg(sidx_hbm, out_hbm, sidx_v, acc_v, sem_s):
        c = lax.axis_index("c")
        s = lax.axis_index("s")
        wid = s * NC + c
        base = wid * CH
        cp_s = pltpu.async_copy(sidx_hbm.at[pl.ds(base, CH)], sidx_v, sem_s)
        _zero_acc(acc_v)
        cp_s.wait()
        ones16 = jnp.ones((16,), jnp.float32)

        def body(i, carry):
            si = sidx_v[pl.ds(i * 16, 16)]
            plsc.addupdate_scatter(acc_v, [si], ones16)
            return carry

        lax.fori_loop(0, CH // 16, body, 0, unroll=8)
        pltpu.sync_copy(acc_v, out_hbm.at[wid])

    return sc_deg


def _make_sc_gs():
    mesh = plsc.VectorSubcoreMesh(core_axis_name="c", subcore_axis_name="s")

    @functools.partial(
        pl.kernel,
        out_type=jax.ShapeDtypeStruct((NW, NPAD), jnp.float32),
        mesh=mesh,
        compiler_params=pltpu.CompilerParams(needs_layout_passes=False),
        scratch_types=[
            pltpu.VMEM((CH,), jnp.int32),
            pltpu.VMEM((CH,), jnp.int32),
            pltpu.VMEM((NPAD,), jnp.float32),
            pltpu.VMEM((NPAD,), jnp.float32),
            pltpu.SemaphoreType.DMA,
            pltpu.SemaphoreType.DMA,
            pltpu.SemaphoreType.DMA,
        ],
    )
    def sc_gs(gidx_hbm, sidx_hbm, table_hbm, out_hbm,
              gidx_v, sidx_v, table_v, acc_v, sem_g, sem_s, sem_t):
        c = lax.axis_index("c")
        s = lax.axis_index("s")
        wid = s * NC + c
        base = wid * CH
        cp_g = pltpu.async_copy(gidx_hbm.at[pl.ds(base, CH)], gidx_v, sem_g)
        cp_s = pltpu.async_copy(sidx_hbm.at[pl.ds(base, CH)], sidx_v, sem_s)
        cp_t = pltpu.async_copy(table_hbm, table_v, sem_t)
        _zero_acc(acc_v)
        cp_g.wait()
        cp_s.wait()
        cp_t.wait()

        def body(i, carry):
            gi = gidx_v[pl.ds(i * 16, 16)]
            val = plsc.load_gather(table_v, [gi])
            si = sidx_v[pl.ds(i * 16, 16)]
            plsc.addupdate_scatter(acc_v, [si], val)
            return carry

        lax.fori_loop(0, CH // 16, body, 0, unroll=8)
        pltpu.sync_copy(acc_v, out_hbm.at[wid])

    return sc_gs


_sc_deg = _make_sc_deg()
_sc_gs = _make_sc_gs()


def _dinv_body(parts_ref, dinv_ref):
    deg = jnp.sum(parts_ref[...], axis=0) + 1.0
    dinv_ref[...] = lax.rsqrt(deg)


def _vw_body(parts_ref, dinv_ref, w_ref, sv_ref):
    dinv = dinv_ref[...]
    v = dinv * (jnp.sum(parts_ref[...], axis=0) + dinv)
    row = lax.broadcasted_iota(jnp.int32, (ROWS, 128), 0)
    col = lax.broadcasted_iota(jnp.int32, (ROWS, 128), 1)
    mask = (row * 128 + col) < N
    sv_ref[...] = jnp.sum(jnp.where(mask, v, 0.0)).reshape(1, 1)
    w_ref[...] = dinv * v


def _head_body(parts_ref, dinv_ref, w_ref, sv_ref, x3_ref,
               state_ref, action_ref,
               Wg1_ref, bg1_ref, Wg2_ref, bg2_ref,
               Wa0_ref, ba0_ref, Wa1_ref, ba1_ref, Wa2_ref, ba2_ref,
               Wc0_ref, bc0_ref, Wc1_ref, bc1_ref, Wc2_ref, bc2_ref,
               alp_ref, sval_ref, ent_ref):
    dinv = dinv_ref[...]
    w = w_ref[...]
    u = dinv * (jnp.sum(parts_ref[...], axis=0) + w)
    row = lax.broadcasted_iota(jnp.int32, (ROWS, 128), 0)
    col = lax.broadcasted_iota(jnp.int32, (ROWS, 128), 1)
    u = jnp.where((row * 128 + col) < N, u, 0.0)
    t = jnp.sum(x3_ref[...] * u[:, :, None], axis=(0, 1)).reshape(1, 128)
    sv = sv_ref[0, 0]
    g1 = jnp.dot(t, Wg1_ref[...], preferred_element_type=jnp.float32) \
        + sv * bg1_ref[...]
    g = jnp.dot(g1, Wg2_ref[...], preferred_element_type=jnp.float32) / N \
        + bg2_ref[...]

    st = state_ref[...]

    def mlp(W0_ref, b0_ref, W1_ref, b1_ref):
        h = jnp.tanh(
            jnp.dot(st, W0_ref[0:128, :], preferred_element_type=jnp.float32)
            + jnp.dot(g, W0_ref[128:256, :], preferred_element_type=jnp.float32)
            + b0_ref[...])
        return jnp.tanh(
            jnp.dot(h, W1_ref[...], preferred_element_type=jnp.float32)
            + b1_ref[...])

    ya = mlp(Wa0_ref, ba0_ref, Wa1_ref, ba1_ref)
    logits = jnp.dot(ya, Wa2_ref[...], preferred_element_type=jnp.float32) \
        + ba2_ref[...]
    m = jnp.max(logits, axis=1, keepdims=True)
    ex = jnp.exp(logits - m)
    ssum = jnp.sum(ex, axis=1, keepdims=True)
    logp = logits - m - jnp.log(ssum)
    probs = ex / ssum
    onehot = lax.broadcasted_iota(jnp.int32, (B, ACT), 1) == action_ref[...]
    alp_ref[...] = jnp.sum(jnp.where(onehot, logp, 0.0), axis=1, keepdims=True)
    ent_ref[...] = -jnp.sum(probs * logp, axis=1, keepdims=True)

    yc = mlp(Wc0_ref, bc0_ref, Wc1_ref, bc1_ref)
    sval_ref[...] = jnp.dot(yc, Wc2_ref[...], preferred_element_type=jnp.float32) \
        + bc2_ref[...]


def kernel(state, action, x, edge_index, W_g1, b_g1, W_g2, b_g2,
           Wa0, ba0, Wa1, ba1, Wa2, ba2, Wc0, bc0, Wc1, bc1, Wc2, bc2):
    src = edge_index[0]
    dst = edge_index[1]

    deg_parts = _sc_deg(dst)

    dinv = pl.pallas_call(
        _dinv_body,
        out_shape=jax.ShapeDtypeStruct((ROWS, 128), jnp.float32),
    )(deg_parts.reshape(NW, ROWS, 128))

    v_parts = _sc_gs(dst, src, dinv.reshape(NPAD))

    w, sv = pl.pallas_call(
        _vw_body,
        out_shape=(jax.ShapeDtypeStruct((ROWS, 128), jnp.float32),
                   jax.ShapeDtypeStruct((1, 1), jnp.float32)),
    )(v_parts.reshape(NW, ROWS, 128), dinv)

    u_parts = _sc_gs(dst, src, w.reshape(NPAD))

    x3 = jnp.pad(x, ((0, NPAD - N), (0, 0))).reshape(ROWS, 128, 128)
    action2 = action.astype(jnp.int32).reshape(B, 1)

    alp, sval, ent = pl.pallas_call(
        _head_body,
        out_shape=(jax.ShapeDtypeStruct((B, 1), jnp.float32),
                   jax.ShapeDtypeStruct((B, 1), jnp.float32),
                   jax.ShapeDtypeStruct((B, 1), jnp.float32)),
    )(u_parts.reshape(NW, ROWS, 128), dinv, w, sv, x3, state, action2,
      W_g1, b_g1, W_g2, b_g2,
      Wa0, ba0, Wa1, ba1, Wa2, ba2,
      Wc0, bc0, Wc1, bc1, Wc2, bc2)

    return (alp[:, 0], sval, ent[:, 0])


# speedup vs baseline: 1.4671x; 1.1252x over previous
import functools

import jax
import jax.numpy as jnp
from jax import lax
from jax.experimental import pallas as pl
from jax.experimental.pallas import tpu as pltpu
from jax.experimental.pallas import tpu_sc as plsc

N = 10000
E = 320000
NPAD = 10240
ROWS = NPAD // 128
NC = 2
NS = 16
NW = NC * NS
CH = E // NW
B = 1024
ACT = 48


def _zero_acc(acc_v):
    zeros16 = jnp.zeros((16,), jnp.float32)

    def zbody(i, carry):
        acc_v[pl.ds(i * 16, 16)] = zeros16
        return carry

    lax.fori_loop(0, NPAD // 16, zbody, 0, unroll=8)


def _make_sc_deg():
    mesh = plsc.VectorSubcoreMesh(core_axis_name="c", subcore_axis_name="s")

    @functools.partial(
        pl.kernel,
        out_type=jax.ShapeDtypeStruct((NW, NPAD), jnp.float32),
        mesh=mesh,
        compiler_params=pltpu.CompilerParams(needs_layout_passes=False),
        scratch_types=[
            pltpu.VMEM((CH,), jnp.int32),
            pltpu.VMEM((NPAD,), jnp.float32),
            pltpu.SemaphoreType.DMA,
        ],
    )
    def sc_deg(sidx_hbm, out_hbm, sidx_v, acc_v, sem_s):
        c = lax.axis_index("c")
        s = lax.axis_index("s")
        wid = s * NC + c
        base = wid * CH
        cp_s = pltpu.async_copy(sidx_hbm.at[pl.ds(base, CH)], sidx_v, sem_s)
        _zero_acc(acc_v)
        cp_s.wait()
        ones16 = jnp.ones((16,), jnp.float32)

        @plsc.parallel_loop(0, CH // 16, unroll=8)
        def _loop(i):
            si = sidx_v[pl.ds(i * 16, 16)]
            plsc.addupdate_scatter(acc_v, [si], ones16)

        pltpu.sync_copy(acc_v, out_hbm.at[wid])

    return sc_deg


def _make_sc_gs():
    mesh = plsc.VectorSubcoreMesh(core_axis_name="c", subcore_axis_name="s")

    @functools.partial(
        pl.kernel,
        out_type=jax.ShapeDtypeStruct((NW, NPAD), jnp.float32),
        mesh=mesh,
        compiler_params=pltpu.CompilerParams(needs_layout_passes=False),
        scratch_types=[
            pltpu.VMEM((CH,), jnp.int32),
            pltpu.VMEM((CH,), jnp.int32),
            pltpu.VMEM((NPAD,), jnp.float32),
            pltpu.VMEM((NPAD,), jnp.float32),
            pltpu.SemaphoreType.DMA,
            pltpu.SemaphoreType.DMA,
            pltpu.SemaphoreType.DMA,
        ],
    )
    def sc_gs(gidx_hbm, sidx_hbm, table_hbm, out_hbm,
              gidx_v, sidx_v, table_v, acc_v, sem_g, sem_s, sem_t):
        c = lax.axis_index("c")
        s = lax.axis_index("s")
        wid = s * NC + c
        base = wid * CH
        cp_g = pltpu.async_copy(gidx_hbm.at[pl.ds(base, CH)], gidx_v, sem_g)
        cp_s = pltpu.async_copy(sidx_hbm.at[pl.ds(base, CH)], sidx_v, sem_s)
        cp_t = pltpu.async_copy(table_hbm, table_v, sem_t)
        _zero_acc(acc_v)
        cp_g.wait()
        cp_s.wait()
        cp_t.wait()

        @plsc.parallel_loop(0, CH // 16, unroll=8)
        def _loop(i):
            gi = gidx_v[pl.ds(i * 16, 16)]
            val = plsc.load_gather(table_v, [gi])
            si = sidx_v[pl.ds(i * 16, 16)]
            plsc.addupdate_scatter(acc_v, [si], val)

        pltpu.sync_copy(acc_v, out_hbm.at[wid])

    return sc_gs


_sc_deg = _make_sc_deg()
_sc_gs = _make_sc_gs()


def _dinv_body(parts_ref, dinv_ref):
    deg = jnp.sum(parts_ref[...], axis=0) + 1.0
    dinv_ref[...] = lax.rsqrt(deg)


def _vw_body(parts_ref, dinv_ref, w_ref, sv_ref):
    dinv = dinv_ref[...]
    v = dinv * (jnp.sum(parts_ref[...], axis=0) + dinv)
    row = lax.broadcasted_iota(jnp.int32, (ROWS, 128), 0)
    col = lax.broadcasted_iota(jnp.int32, (ROWS, 128), 1)
    mask = (row * 128 + col) < N
    sv_ref[...] = jnp.sum(jnp.where(mask, v, 0.0)).reshape(1, 1)
    w_ref[...] = dinv * v


def _head_body(parts_ref, dinv_ref, w_ref, sv_ref, x3_ref,
               state_ref, action_ref,
               Wg1_ref, bg1_ref, Wg2_ref, bg2_ref,
               Wa0_ref, ba0_ref, Wa1_ref, ba1_ref, Wa2_ref, ba2_ref,
               Wc0_ref, bc0_ref, Wc1_ref, bc1_ref, Wc2_ref, bc2_ref,
               alp_ref, sval_ref, ent_ref):
    dinv = dinv_ref[...]
    w = w_ref[...]
    u = dinv * (jnp.sum(parts_ref[...], axis=0) + w)
    row = lax.broadcasted_iota(jnp.int32, (ROWS, 128), 0)
    col = lax.broadcasted_iota(jnp.int32, (ROWS, 128), 1)
    u = jnp.where((row * 128 + col) < N, u, 0.0)
    t = jnp.sum(x3_ref[...] * u[:, :, None], axis=(0, 1)).reshape(1, 128)
    sv = sv_ref[0, 0]
    g1 = jnp.dot(t, Wg1_ref[...], preferred_element_type=jnp.float32) \
        + sv * bg1_ref[...]
    g = jnp.dot(g1, Wg2_ref[...], preferred_element_type=jnp.float32) / N \
        + bg2_ref[...]

    st = state_ref[...]

    def mlp(W0_ref, b0_ref, W1_ref, b1_ref):
        h = jnp.tanh(
            jnp.dot(st, W0_ref[0:128, :], preferred_element_type=jnp.float32)
            + jnp.dot(g, W0_ref[128:256, :], preferred_element_type=jnp.float32)
            + b0_ref[...])
        return jnp.tanh(
            jnp.dot(h, W1_ref[...], preferred_element_type=jnp.float32)
            + b1_ref[...])

    ya = mlp(Wa0_ref, ba0_ref, Wa1_ref, ba1_ref)
    logits = jnp.dot(ya, Wa2_ref[...], preferred_element_type=jnp.float32) \
        + ba2_ref[...]
    m = jnp.max(logits, axis=1, keepdims=True)
    ex = jnp.exp(logits - m)
    ssum = jnp.sum(ex, axis=1, keepdims=True)
    logp = logits - m - jnp.log(ssum)
    probs = ex / ssum
    onehot = lax.broadcasted_iota(jnp.int32, (B, ACT), 1) == action_ref[...]
    alp_ref[...] = jnp.sum(jnp.where(onehot, logp, 0.0), axis=1, keepdims=True)
    ent_ref[...] = -jnp.sum(probs * logp, axis=1, keepdims=True)

    yc = mlp(Wc0_ref, bc0_ref, Wc1_ref, bc1_ref)
    sval_ref[...] = jnp.dot(yc, Wc2_ref[...], preferred_element_type=jnp.float32) \
        + bc2_ref[...]


def kernel(state, action, x, edge_index, W_g1, b_g1, W_g2, b_g2,
           Wa0, ba0, Wa1, ba1, Wa2, ba2, Wc0, bc0, Wc1, bc1, Wc2, bc2):
    src = edge_index[0]
    dst = edge_index[1]

    deg_parts = _sc_deg(dst)

    dinv = pl.pallas_call(
        _dinv_body,
        out_shape=jax.ShapeDtypeStruct((ROWS, 128), jnp.float32),
    )(deg_parts.reshape(NW, ROWS, 128))

    v_parts = _sc_gs(dst, src, dinv.reshape(NPAD))

    w, sv = pl.pallas_call(
        _vw_body,
        out_shape=(jax.ShapeDtypeStruct((ROWS, 128), jnp.float32),
                   jax.ShapeDtypeStruct((1, 1), jnp.float32)),
    )(v_parts.reshape(NW, ROWS, 128), dinv)

    u_parts = _sc_gs(dst, src, w.reshape(NPAD))

    x3 = jnp.pad(x, ((0, NPAD - N), (0, 0))).reshape(ROWS, 128, 128)
    action2 = action.astype(jnp.int32).reshape(B, 1)

    alp, sval, ent = pl.pallas_call(
        _head_body,
        out_shape=(jax.ShapeDtypeStruct((B, 1), jnp.float32),
                   jax.ShapeDtypeStruct((B, 1), jnp.float32),
                   jax.ShapeDtypeStruct((B, 1), jnp.float32)),
    )(u_parts.reshape(NW, ROWS, 128), dinv, w, sv, x3, state, action2,
      W_g1, b_g1, W_g2, b_g2,
      Wa0, ba0, Wa1, ba1, Wa2, ba2,
      Wc0, bc0, Wc1, bc1, Wc2, bc2)

    return (alp[:, 0], sval, ent[:, 0])
